# baseline (speedup 1.0000x reference)
import functools

import jax
import jax.numpy as jnp
from jax import lax
from jax.experimental import pallas as pl
from jax.experimental.pallas import tpu as pltpu
from jax.experimental.pallas import tpu_sc as plsc

_D = 64
_NUM_CORES = 2
_NUM_SUBCORES = 16
_NUM_WORKERS = _NUM_CORES * _NUM_SUBCORES
_BBLK = 128
_NG = 3
_NT = 2


@functools.lru_cache(maxsize=None)
def _make_gather(batch, hist):
    b_per_w = batch // _NUM_WORKERS
    n_bblk = b_per_w // _BBLK
    n_units = hist * n_bblk
    mesh = plsc.VectorSubcoreMesh(core_axis_name="c", subcore_axis_name="s")

    def body(table_hbm, idxt_hbm, out_hbm, idx_v, g_v, t_v, gsem, osem):
        wid = lax.axis_index("s") * _NUM_CORES + lax.axis_index("c")
        b0 = wid * b_per_w
        pltpu.sync_copy(idxt_hbm.at[:, pl.ds(b0, b_per_w)], idx_v)

        iota = lax.iota(jnp.int32, 16)
        idx_dh = [(iota + k * 16) >> 3 for k in range(4)]
        idx_dl = [(iota + k * 16) & 7 for k in range(4)]

        def gather(u, slot):
            h = u // n_bblk
            bh = u % n_bblk
            return pltpu.make_async_copy(
                table_hbm.at[idx_v.at[h, pl.ds(bh * _BBLK, _BBLK)]],
                g_v.at[slot],
                gsem,
            )

        def out_copy(u, slot):
            h = u // n_bblk
            bhg = wid * n_bblk + u % n_bblk
            return pltpu.make_async_copy(
                t_v.at[slot, :, :, pl.ds(0, _BBLK)],
                out_hbm.at[h, :, bhg],
                osem,
            )

        def transpose(gslot, tslot):
            @plsc.parallel_loop(0, _BBLK, unroll=8)
            def bloop(b):
                idx_b = jnp.full((16,), b, jnp.int32)
                for k in range(4):
                    vec = g_v[gslot, b, pl.ds(k * 16, 16)]
                    plsc.store_scatter(
                        t_v.at[tslot], [idx_dh[k], idx_dl[k], idx_b], vec
                    )

        for s in range(_NG - 1):
            gather(s, s).start()

        def step(u, carry):
            gather(u, u % _NG).wait()

            @pl.when(u >= _NT)
            def _():
                out_copy(u - _NT, u % _NT).wait()

            transpose(u % _NG, u % _NT)
            out_copy(u, u % _NT).start()

            @pl.when(u + _NG - 1 < n_units)
            def _():
                gather(u + _NG - 1, (u + _NG - 1) % _NG).start()

            return carry

        lax.fori_loop(0, n_units, step, 0)
        for u in range(n_units - _NT, n_units):
            out_copy(u, u % _NT).wait()

    return pl.kernel(
        body,
        mesh=mesh,
        out_type=jax.ShapeDtypeStruct(
            (hist, 8, batch // 128, 8, 128), jnp.float32
        ),
        scratch_types=[
            pltpu.VMEM((hist, b_per_w), jnp.int32),
            pltpu.VMEM((_NG, _BBLK, _D), jnp.float32),
            pltpu.VMEM((_NT, 8, 8, _BBLK + 5), jnp.float32),
            pltpu.SemaphoreType.DMA,
            pltpu.SemaphoreType.DMA,
        ],
        compiler_params=pltpu.CompilerParams(
            use_tc_tiling_on_sc=False, needs_layout_passes=False
        ),
    )


def kernel(s0, ss, z, embedding):
    b, h = z.shape
    idxt = z.astype(jnp.int32).T
    out = _make_gather(b, h)(embedding, idxt)
    q = jnp.transpose(out, (0, 1, 3, 2, 4)).reshape(h, _D, b)
    return jnp.transpose(q, (2, 0, 1))


# speedup vs baseline: 1.0471x; 1.0471x over previous
import functools

import jax
import jax.numpy as jnp
from jax import lax
from jax.experimental import pallas as pl
from jax.experimental.pallas import tpu as pltpu
from jax.experimental.pallas import tpu_sc as plsc

_D = 64
_NUM_CORES = 2
_NUM_SUBCORES = 16
_NUM_WORKERS = _NUM_CORES * _NUM_SUBCORES
_BBLK = 128
_NG = 4
_NT = 2


@functools.lru_cache(maxsize=None)
def _make_gather(batch, hist):
    b_per_w = batch // _NUM_WORKERS
    n_bblk = b_per_w // _BBLK
    n_units = hist * n_bblk
    mesh = plsc.VectorSubcoreMesh(core_axis_name="c", subcore_axis_name="s")

    def body(table_hbm, idxt_hbm, out_hbm, idx_v, g_v, t_v, gsem, osem):
        wid = lax.axis_index("s") * _NUM_CORES + lax.axis_index("c")
        b0 = wid * b_per_w
        pltpu.sync_copy(idxt_hbm.at[:, pl.ds(b0, b_per_w)], idx_v)

        iota = lax.iota(jnp.int32, 16)
        idx_dh = [(iota + k * 16) >> 3 for k in range(4)]
        idx_dl = [(iota + k * 16) & 7 for k in range(4)]

        def gather(u, slot):
            h = u // n_bblk
            bh = u % n_bblk
            return pltpu.make_async_copy(
                table_hbm.at[idx_v.at[h, pl.ds(bh * _BBLK, _BBLK)]],
                g_v.at[slot],
                gsem,
            )

        def out_copy(u, slot):
            h = u // n_bblk
            bhg = wid * n_bblk + u % n_bblk
            return pltpu.make_async_copy(
                t_v.at[slot, :, :, pl.ds(0, _BBLK)],
                out_hbm.at[h, :, bhg],
                osem,
            )

        def transpose(gslot, tslot):
            @plsc.parallel_loop(0, _BBLK, unroll=8)
            def bloop(b):
                idx_b = jnp.full((16,), b, jnp.int32)
                for k in range(4):
                    vec = g_v[gslot, b, pl.ds(k * 16, 16)]
                    plsc.store_scatter(
                        t_v.at[tslot], [idx_dh[k], idx_dl[k], idx_b], vec
                    )

        for s in range(_NG - 1):
            gather(s, s).start()

        def step(u, carry):
            gather(u, u % _NG).wait()

            @pl.when(u >= _NT)
            def _():
                out_copy(u - _NT, u % _NT).wait()

            transpose(u % _NG, u % _NT)
            out_copy(u, u % _NT).start()

            @pl.when(u + _NG - 1 < n_units)
            def _():
                gather(u + _NG - 1, (u + _NG - 1) % _NG).start()

            return carry

        lax.fori_loop(0, n_units, step, 0)
        for u in range(n_units - _NT, n_units):
            out_copy(u, u % _NT).wait()

    return pl.kernel(
        body,
        mesh=mesh,
        out_type=jax.ShapeDtypeStruct(
            (hist, 8, batch // 128, 8, 128), jnp.float32
        ),
        scratch_types=[
            pltpu.VMEM((hist, b_per_w), jnp.int32),
            pltpu.VMEM((_NG, _BBLK, _D), jnp.float32),
            pltpu.VMEM((_NT, 8, 8, _BBLK + 5), jnp.float32),
            pltpu.SemaphoreType.DMA,
            pltpu.SemaphoreType.DMA,
        ],
        compiler_params=pltpu.CompilerParams(
            use_tc_tiling_on_sc=False, needs_layout_passes=False
        ),
    )


def kernel(s0, ss, z, embedding):
    b, h = z.shape
    idxt = z.astype(jnp.int32).T
    out = _make_gather(b, h)(embedding, idxt)
    q = jnp.transpose(out, (0, 1, 3, 2, 4)).reshape(h, _D, b)
    return jnp.transpose(q, (2, 0, 1))


# speedup vs baseline: 1.0535x; 1.0062x over previous
import functools

import jax
import jax.numpy as jnp
from jax import lax
from jax.experimental import pallas as pl
from jax.experimental.pallas import tpu as pltpu
from jax.experimental.pallas import tpu_sc as plsc

_D = 64
_NUM_CORES = 2
_NUM_SUBCORES = 16
_NUM_WORKERS = _NUM_CORES * _NUM_SUBCORES
_BBLK = 128
_NG = 6
_NT = 3


@functools.lru_cache(maxsize=None)
def _make_gather(batch, hist):
    b_per_w = batch // _NUM_WORKERS
    n_bblk = b_per_w // _BBLK
    n_units = hist * n_bblk
    mesh = plsc.VectorSubcoreMesh(core_axis_name="c", subcore_axis_name="s")

    def body(table_hbm, idxt_hbm, out_hbm, idx_v, g_v, t_v, gsem, osem):
        wid = lax.axis_index("s") * _NUM_CORES + lax.axis_index("c")
        b0 = wid * b_per_w
        pltpu.sync_copy(idxt_hbm.at[:, pl.ds(b0, b_per_w)], idx_v)

        iota = lax.iota(jnp.int32, 16)
        idx_dh = [(iota + k * 16) >> 3 for k in range(4)]
        idx_dl = [(iota + k * 16) & 7 for k in range(4)]

        def gather(u, slot):
            h = u // n_bblk
            bh = u % n_bblk
            return pltpu.make_async_copy(
                table_hbm.at[idx_v.at[h, pl.ds(bh * _BBLK, _BBLK)]],
                g_v.at[slot],
                gsem,
            )

        def out_copy(u, slot):
            h = u // n_bblk
            bhg = wid * n_bblk + u % n_bblk
            return pltpu.make_async_copy(
                t_v.at[slot, :, :, pl.ds(0, _BBLK)],
                out_hbm.at[h, :, bhg],
                osem,
            )

        def transpose(gslot, tslot):
            @plsc.parallel_loop(0, _BBLK, unroll=8)
            def bloop(b):
                idx_b = jnp.full((16,), b, jnp.int32)
                for k in range(4):
                    vec = g_v[gslot, b, pl.ds(k * 16, 16)]
                    plsc.store_scatter(
                        t_v.at[tslot], [idx_dh[k], idx_dl[k], idx_b], vec
                    )

        for s in range(_NG - 1):
            gather(s, s).start()

        def step(u, carry):
            gather(u, u % _NG).wait()

            @pl.when(u >= _NT)
            def _():
                out_copy(u - _NT, u % _NT).wait()

            transpose(u % _NG, u % _NT)
            out_copy(u, u % _NT).start()

            @pl.when(u + _NG - 1 < n_units)
            def _():
                gather(u + _NG - 1, (u + _NG - 1) % _NG).start()

            return carry

        lax.fori_loop(0, n_units, step, 0)
        for u in range(n_units - _NT, n_units):
            out_copy(u, u % _NT).wait()

    return pl.kernel(
        body,
        mesh=mesh,
        out_type=jax.ShapeDtypeStruct(
            (hist, 8, batch // 128, 8, 128), jnp.float32
        ),
        scratch_types=[
            pltpu.VMEM((hist, b_per_w), jnp.int32),
            pltpu.VMEM((_NG, _BBLK, _D), jnp.float32),
            pltpu.VMEM((_NT, 8, 8, _BBLK + 5), jnp.float32),
            pltpu.SemaphoreType.DMA,
            pltpu.SemaphoreType.DMA,
        ],
        compiler_params=pltpu.CompilerParams(
            use_tc_tiling_on_sc=False, needs_layout_passes=False
        ),
    )


def kernel(s0, ss, z, embedding):
    b, h = z.shape
    idxt = z.astype(jnp.int32).T
    out = _make_gather(b, h)(embedding, idxt)
    q = jnp.transpose(out, (0, 1, 3, 2, 4)).reshape(h, _D, b)
    return jnp.transpose(q, (2, 0, 1))
